# baseline (speedup 1.0000x reference)
import functools

import jax
import jax.numpy as jnp
from jax import lax
from jax.experimental import pallas as pl
from jax.experimental.pallas import tpu as pltpu
from jax.experimental.pallas import tpu_sc as plsc

N = 10000
E = 160000
D = 256
L = 3
NEXP = 8
H = 64
T = 12
NG = 128
FX = 9
FE = 3
HD = 128
NSUB = 16
NB = 25
BR = 400


_NODE_CHUNK = 16
_ROWS_PER_CHUNK = _NODE_CHUNK * FX


def _enc_body(xid_hbm, atab_hbm, h_hbm, idxv, gbuf, obuf, sem):
    c = lax.axis_index("c")
    s = lax.axis_index("s")
    coff = c * (FX * 100)
    base_n = s * 640
    nch = jnp.where(s < NSUB - 1, 40, 25)

    def chunk(k, _):
        nb = base_n + k * _NODE_CHUNK
        pltpu.sync_copy(xid_hbm.at[pl.ds(nb * FX, _ROWS_PER_CHUNK)], idxv)
        for j in range(_ROWS_PER_CHUNK // 16):
            idxv[pl.ds(16 * j, 16)] = idxv[pl.ds(16 * j, 16)] + coff
        pltpu.async_copy(atab_hbm.at[idxv], gbuf, sem).wait()

        def node(n, _):
            for j in range(HD // 16):
                acc = gbuf[n * FX, pl.ds(16 * j, 16)]
                for f in range(1, FX):
                    acc = acc + gbuf[n * FX + f, pl.ds(16 * j, 16)]
                obuf[n, pl.ds(16 * j, 16)] = acc
            return 0

        lax.fori_loop(0, _NODE_CHUNK, node, 0)
        pltpu.sync_copy(obuf, h_hbm.at[pl.ds(c * N + nb, _NODE_CHUNK)])
        return 0

    lax.fori_loop(0, nch, chunk, 0)


def _encode_sc(xid, atab_cat):
    mesh = plsc.VectorSubcoreMesh(core_axis_name="c", subcore_axis_name="s")
    return pl.kernel(
        _enc_body,
        out_type=jax.ShapeDtypeStruct((2 * N, HD), jnp.float32),
        mesh=mesh,
        scratch_types=[
            pltpu.VMEM((_ROWS_PER_CHUNK,), jnp.int32),
            pltpu.VMEM((_ROWS_PER_CHUNK, HD), jnp.float32),
            pltpu.VMEM((_NODE_CHUNK, HD), jnp.float32),
            pltpu.SemaphoreType.DMA,
        ],
    )(xid, atab_cat)


_ECH = 80
_NCHUNK = E // NSUB // _ECH
_ZROWS = 80


def _msg_body(h_hbm, ctab_hbm, eidx_hbm, agg_hbm,
              pidx0, pidx1, dbuf0, dbuf1, hbuf0, hbuf1, cbuf0, cbuf1, accum,
              semi0, semi1, semh0, semh1, semc0, semc1, sems0, sems1):
    c = lax.axis_index("c")
    s = lax.axis_index("s")
    cN = c * N
    c512 = c * 512
    tbase = s * _NCHUNK * 3 * _ECH

    bufs = ((pidx0, dbuf0, hbuf0, cbuf0, semi0, semh0, semc0, sems0),
            (pidx1, dbuf1, hbuf1, cbuf1, semi1, semh1, semc1, sems1))

    def issue_idx(k, slot):
        pi, _, _, _, si, _, _, _ = bufs[slot]
        pltpu.async_copy(
            eidx_hbm.at[pl.ds(tbase + k * 3 * _ECH, 3 * _ECH)], pi, si)

    def wait_idx(slot):
        pi, _, _, _, si, _, _, _ = bufs[slot]
        pltpu.make_async_copy(eidx_hbm.at[pl.ds(0, 3 * _ECH)], pi, si).wait()

    def prep_and_gather(slot):
        pi, db, hb, cb, _, sh, sc_, _ = bufs[slot]
        for j in range(_ECH // 16):
            pi[pl.ds(16 * j, 16)] = pi[pl.ds(16 * j, 16)] + cN
            pi[pl.ds(_ECH + 16 * j, 16)] = pi[pl.ds(_ECH + 16 * j, 16)] + c512
            db[pl.ds(16 * j, 16)] = pi[pl.ds(2 * _ECH + 16 * j, 16)]
        pltpu.async_copy(h_hbm.at[pi.at[pl.ds(0, _ECH)]], hb, sh)
        pltpu.async_copy(ctab_hbm.at[pi.at[pl.ds(_ECH, _ECH)]], cb, sc_)

    def wait_gathers(slot):
        pi, _, hb, cb, _, sh, sc_, _ = bufs[slot]
        pltpu.make_async_copy(h_hbm.at[pi.at[pl.ds(0, _ECH)]], hb, sh).wait()
        pltpu.make_async_copy(
            ctab_hbm.at[pi.at[pl.ds(_ECH, _ECH)]], cb, sc_).wait()

    def wait_scatter(slot):
        _, db, hb, _, _, _, _, ss = bufs[slot]
        pltpu.make_async_copy(hb, accum.at[db], ss).wait()

    rbase = s * 640
    nz = jnp.where(s < NSUB - 1, 8, 5)

    issue_idx(0, 0)

    def zrow(r, _):
        for j in range(HD // 16):
            hbuf0[r, pl.ds(16 * j, 16)] = jnp.zeros((16,), jnp.float32)
        return 0

    lax.fori_loop(0, _ZROWS, zrow, 0)

    def zcopy(r, _):
        pltpu.sync_copy(hbuf0, accum.at[pl.ds(rbase + r * _ZROWS, _ZROWS)])
        return 0

    lax.fori_loop(0, nz, zcopy, 0)

    wait_idx(0)
    prep_and_gather(0)
    issue_idx(1, 1)
    plsc.subcore_barrier()

    def do_chunk(k, slot):
        pi, db, hb, cb, _, _, _, ss = bufs[slot]
        nxt = 1 - slot
        wait_gathers(slot)

        @pl.when(k + 1 < _NCHUNK)
        def _():
            wait_idx(nxt)

            @pl.when(k >= 1)
            def _():
                wait_scatter(nxt)
            prep_and_gather(nxt)

        @pl.when(k + 2 < _NCHUNK)
        def _():
            issue_idx(k + 2, slot)

        def edge(e, _):
            for j in range(HD // 16):
                hv = hb[e, pl.ds(16 * j, 16)]
                cv = cb[e, pl.ds(16 * j, 16)]
                hb[e, pl.ds(16 * j, 16)] = jnp.maximum(hv + cv, 0.0)
            return 0

        lax.fori_loop(0, _ECH, edge, 0, unroll=2)
        pltpu.async_copy(hb, accum.at[db], ss, add=True)

    def chunk(k, _):
        @pl.when(k % 2 == 0)
        def _():
            do_chunk(k, 0)

        @pl.when(k % 2 == 1)
        def _():
            do_chunk(k, 1)
        return 0

    lax.fori_loop(0, _NCHUNK, chunk, 0)
    wait_scatter((_NCHUNK - 2) % 2)
    wait_scatter((_NCHUNK - 1) % 2)
    plsc.subcore_barrier()

    def wcopy(r, _):
        pltpu.sync_copy(accum.at[pl.ds(rbase + r * _ZROWS, _ZROWS)],
                        agg_hbm.at[pl.ds(c * N + rbase + r * _ZROWS, _ZROWS)])
        return 0

    lax.fori_loop(0, nz, wcopy, 0)


def _msg_sc(h_cat, ctab_cat, eidx):
    mesh = plsc.VectorSubcoreMesh(core_axis_name="c", subcore_axis_name="s")
    return pl.kernel(
        _msg_body,
        out_type=jax.ShapeDtypeStruct((2 * N, HD), jnp.float32),
        mesh=mesh,
        scratch_types=[
            pltpu.VMEM((3 * _ECH,), jnp.int32),
            pltpu.VMEM((3 * _ECH,), jnp.int32),
            pltpu.VMEM((_ECH,), jnp.int32),
            pltpu.VMEM((_ECH,), jnp.int32),
            pltpu.VMEM((_ECH, HD), jnp.float32),
            pltpu.VMEM((_ECH, HD), jnp.float32),
            pltpu.VMEM((_ECH, HD), jnp.float32),
            pltpu.VMEM((_ECH, HD), jnp.float32),
            pltpu.VMEM_SHARED((N, HD), jnp.float32),
        ] + [pltpu.SemaphoreType.DMA] * 8,
    )(h_cat, ctab_cat, eidx)


def _ctab_body(be_ref, out_ref):
    rr = lax.broadcasted_iota(jnp.int32, (512, 8), 0)
    cc = lax.broadcasted_iota(jnp.int32, (512, 8), 1)
    a0 = ((rr // 64) == cc).astype(jnp.float32)
    a1 = (((rr // 8) % 8) == cc).astype(jnp.float32)
    a2 = ((rr % 8) == cc).astype(jnp.float32)
    c = (jnp.dot(a0, be_ref[0], preferred_element_type=jnp.float32, precision=lax.Precision.HIGHEST)
         + jnp.dot(a1, be_ref[1], preferred_element_type=jnp.float32, precision=lax.Precision.HIGHEST)
         + jnp.dot(a2, be_ref[2], preferred_element_type=jnp.float32, precision=lax.Precision.HIGHEST))
    out_ref[...] = jnp.concatenate([c[:, :HD], c[:, HD:]], axis=0)


def _ctab_tc(bond_emb):
    return pl.pallas_call(
        _ctab_body,
        out_shape=jax.ShapeDtypeStruct((1024, HD), jnp.float32),
    )(bond_emb)


def _phase_a_body(h0_ref, h1_ref, a0_ref, a1_ref, gw_ref, w1_ref, w2_ref,
                  batch_ref, hn_ref, psum_ref, cnts_ref, acc_ref):
    i = pl.program_id(0)
    z = jnp.concatenate(
        [h0_ref[...] + a0_ref[...], h1_ref[...] + a1_ref[...]], axis=1)
    zb = z.astype(jnp.bfloat16)
    logits = jnp.dot(zb, gw_ref[...], preferred_element_type=jnp.float32)
    lane = lax.broadcasted_iota(jnp.int32, (1, 128), 1)
    neg = jnp.float32(-1e30)
    logits = logits + jnp.where(lane < NEXP, 0.0, neg)
    v1 = jnp.max(logits, axis=1, keepdims=True)
    i1 = jnp.min(jnp.where(logits == v1, lane, 10 ** 9), axis=1, keepdims=True)
    l2 = jnp.where(lane == i1, neg, logits)
    v2 = jnp.max(l2, axis=1, keepdims=True)
    i2 = jnp.min(jnp.where(l2 == v2, lane, 10 ** 9), axis=1, keepdims=True)
    e2 = jnp.exp(v2 - v1)
    p1 = 1.0 / (1.0 + e2)
    p2 = e2 * p1
    gates = jnp.where(lane == i1, p1, 0.0) + jnp.where(lane == i2, p2, 0.0)
    ex = jnp.exp(logits - v1)
    sm = ex / jnp.sum(ex, axis=1, keepdims=True)

    h1 = jnp.maximum(
        jnp.dot(zb, w1_ref[...], preferred_element_type=jnp.float32), 0.0)
    hn = jnp.zeros((BR, D), jnp.float32)
    for e in range(NEXP):
        h2e = jnp.dot(h1[:, e * H:(e + 1) * H].astype(jnp.bfloat16),
                      w2_ref[e * H:(e + 1) * H, :],
                      preferred_element_type=jnp.float32)
        ge = gates[:, e:e + 1].astype(jnp.bfloat16).astype(jnp.float32)
        hn = hn + ge * h2e.astype(jnp.bfloat16).astype(jnp.float32)
    hn_ref[...] = hn

    b = batch_ref[0, 0]
    onehot = (b[:, None] == lax.broadcasted_iota(jnp.int32, (BR, NG), 1)
              ).astype(jnp.float32)
    ps = lax.dot_general(onehot, hn, (((0,), (0,)), ((), ())),
                         preferred_element_type=jnp.float32, precision=lax.Precision.HIGHEST)
    cnt = lax.dot_general(onehot, jnp.ones((BR, 8), jnp.float32),
                          (((0,), (0,)), ((), ())),
                          preferred_element_type=jnp.float32, precision=lax.Precision.HIGHEST)
    pad = jnp.zeros((1, 128), jnp.float32)
    acc = jnp.concatenate([
        jnp.sum(hn, axis=0, keepdims=True),
        jnp.sum(hn * hn, axis=0, keepdims=True),
        jnp.concatenate([jnp.sum(gates, axis=0, keepdims=True), pad], axis=1),
        jnp.concatenate([jnp.sum(sm, axis=0, keepdims=True), pad], axis=1),
        jnp.zeros((4, 256), jnp.float32),
    ], axis=0)

    @pl.when(i == 0)
    def _():
        psum_ref[...] = ps
        cnts_ref[...] = cnt
        acc_ref[...] = acc

    @pl.when(i > 0)
    def _():
        psum_ref[...] = psum_ref[...] + ps
        cnts_ref[...] = cnts_ref[...] + cnt
        acc_ref[...] = acc_ref[...] + acc


def _phase_a(h_cat, agg_cat, gw_pad, w1cat, w2cat, batch3):
    return pl.pallas_call(
        _phase_a_body,
        grid=(NB,),
        in_specs=[
            pl.BlockSpec((BR, HD), lambda i: (i, 0)),
            pl.BlockSpec((BR, HD), lambda i: (NB + i, 0)),
            pl.BlockSpec((BR, HD), lambda i: (i, 0)),
            pl.BlockSpec((BR, HD), lambda i: (NB + i, 0)),
            pl.BlockSpec((D, 128), lambda i: (0, 0)),
            pl.BlockSpec((D, NEXP * H), lambda i: (0, 0)),
            pl.BlockSpec((NEXP * H, D), lambda i: (0, 0)),
            pl.BlockSpec((1, 1, BR), lambda i: (i, 0, 0)),
        ],
        out_specs=[
            pl.BlockSpec((BR, D), lambda i: (i, 0)),
            pl.BlockSpec((NG, D), lambda i: (0, 0)),
            pl.BlockSpec((NG, 8), lambda i: (0, 0)),
            pl.BlockSpec((8, D), lambda i: (0, 0)),
        ],
        out_shape=[
            jax.ShapeDtypeStruct((N, D), jnp.float32),
            jax.ShapeDtypeStruct((NG, D), jnp.float32),
            jax.ShapeDtypeStruct((NG, 8), jnp.float32),
            jax.ShapeDtypeStruct((8, D), jnp.float32),
        ],
    )(h_cat, h_cat, agg_cat, agg_cat, gw_pad, w1cat, w2cat, batch3)


def _phase_b_body(hn_ref, acc_ref, g_ref, b_ref, out_ref):
    mu = acc_ref[0:1, :] / N
    var = acc_ref[1:2, :] / N - mu * mu
    scale = g_ref[0] * lax.rsqrt(var + 1e-5)
    shift = b_ref[0] - mu * scale
    out_ref[...] = jnp.maximum(hn_ref[...] * scale + shift, 0.0)


def _phase_b(hn, acc, bn_g2, bn_b2):
    return pl.pallas_call(
        _phase_b_body,
        grid=(2, NB),
        in_specs=[
            pl.BlockSpec((BR, HD), lambda h, j: (j, h)),
            pl.BlockSpec((8, HD), lambda h, j: (0, h)),
            pl.BlockSpec((1, 1, HD), lambda h, j: (h, 0, 0)),
            pl.BlockSpec((1, 1, HD), lambda h, j: (h, 0, 0)),
        ],
        out_specs=pl.BlockSpec((BR, HD), lambda h, j: (h * NB + j, 0)),
        out_shape=jax.ShapeDtypeStruct((2 * N, HD), jnp.float32),
    )(hn, acc, bn_g2, bn_b2)


def _final_body(ps0_ref, ps1_ref, ps2_ref, cnts_ref, ac0_ref, ac1_ref, ac2_ref,
                pw_ref, pb_ref, ow_ref, ob_ref, y_ref, aux_ref):
    cnt = jnp.maximum(cnts_ref[:, 0:1], 1.0)
    ys = []
    for l, ps_ref in enumerate((ps0_ref, ps1_ref, ps2_ref)):
        pooled = (ps_ref[...] / cnt).astype(jnp.bfloat16)
        ys.append(jnp.dot(pooled, pw_ref[l], preferred_element_type=jnp.float32)
                  + pb_ref[l])
    aux = jnp.float32(0.0)
    for ac_ref in (ac0_ref, ac1_ref, ac2_ref):
        aux = aux + jnp.sum(ac_ref[2:3, :] * ac_ref[3:4, :])
    aux = aux * (NEXP / float(N * N))
    ysb = [y.astype(jnp.bfloat16).astype(jnp.float32) for y in ys]
    owf = ow_ref[...].astype(jnp.float32)
    for u in range(T):
        yu = (ysb[0] * owf[u:u + 1, 0:1] + ysb[1] * owf[u:u + 1, 1:2]
              + ysb[2] * owf[u:u + 1, 2:3] + ob_ref[0:1, u:u + 1])
        y_ref[u] = 1.0 / (1.0 + jnp.exp(-yu))
    aux_ref[...] = jnp.full((8, 128), aux, jnp.float32)


def _final(psums, cnts, accs, pred_w, pred_b3, out_w, out_b2):
    return pl.pallas_call(
        _final_body,
        out_shape=[
            jax.ShapeDtypeStruct((T, NG, T), jnp.float32),
            jax.ShapeDtypeStruct((8, 128), jnp.float32),
        ],
    )(psums[0], psums[1], psums[2], cnts, accs[0], accs[1], accs[2],
      pred_w, pred_b3, out_w, out_b2)


def kernel(x, edge_index, edge_attr, batch, atom_emb, bond_emb, gate_w, w1,
           w2, bn_g, bn_b, pred_w, pred_b, out_w, out_b):
    x = x.astype(jnp.int32)
    edge_attr = edge_attr.astype(jnp.int32)
    src = edge_index[0].astype(jnp.int32)
    dst = edge_index[1].astype(jnp.int32)
    xid = (x + jnp.arange(FX, dtype=jnp.int32)[None, :] * 100).reshape(-1)
    cidx = edge_attr[:, 0] * 64 + edge_attr[:, 1] * 8 + edge_attr[:, 2]
    eidx = jnp.stack([src.reshape(-1, _ECH), cidx.reshape(-1, _ECH),
                      dst.reshape(-1, _ECH)], axis=1).reshape(-1)
    atab_cat = jnp.transpose(
        atom_emb.reshape(FX * 100, 2, HD), (1, 0, 2)).reshape(2 * FX * 100, HD)
    batch3 = batch.astype(jnp.int32).reshape(NB, 1, BR)
    gw_pad = jnp.concatenate(
        [gate_w, jnp.zeros((L, D, 128 - NEXP), jnp.float32)],
        axis=2).astype(jnp.bfloat16)
    w1cat = jnp.transpose(w1, (0, 2, 1, 3)).reshape(L, D, NEXP * H).astype(jnp.bfloat16)
    w2cat = w2.reshape(L, NEXP * H, D).astype(jnp.bfloat16)

    ctab_cat = _ctab_tc(bond_emb)
    h_cat = _encode_sc(xid, atab_cat)

    psums, accs = [], []
    cnts = None
    for l in range(L):
        agg_cat = _msg_sc(h_cat, ctab_cat, eidx)
        hn, psum, cnt, acc = _phase_a(
            h_cat, agg_cat, gw_pad[l], w1cat[l], w2cat[l], batch3)
        psums.append(psum)
        accs.append(acc)
        if cnts is None:
            cnts = cnt
        if l < L - 1:
            h_cat = _phase_b(hn, acc, bn_g[l].reshape(2, 1, HD),
                             bn_b[l].reshape(2, 1, HD))

    y3, aux = _final(psums, cnts, accs, pred_w.astype(jnp.bfloat16),
                     pred_b.reshape(L, 1, T), out_w.astype(jnp.bfloat16),
                     out_b.reshape(1, T))
    y = jnp.transpose(y3, (1, 2, 0))
    return y, aux[0, 0]


# speedup vs baseline: 1.4951x; 1.4951x over previous
import functools

import jax
import jax.numpy as jnp
from jax import lax
from jax.experimental import pallas as pl
from jax.experimental.pallas import tpu as pltpu
from jax.experimental.pallas import tpu_sc as plsc

N = 10000
E = 160000
D = 256
L = 3
NEXP = 8
H = 64
T = 12
NG = 128
FX = 9
FE = 3
HD = 128
NSUB = 16
NB = 25
BR = 400


_NODE_CHUNK = 16
_ROWS_PER_CHUNK = _NODE_CHUNK * FX


def _enc_body(xid_hbm, atab_hbm, h_hbm, idxv, gbuf, obuf, sem):
    c = lax.axis_index("c")
    s = lax.axis_index("s")
    coff = c * (FX * 100)
    base_n = s * 640
    nch = jnp.where(s < NSUB - 1, 40, 25)

    def chunk(k, _):
        nb = base_n + k * _NODE_CHUNK
        pltpu.sync_copy(xid_hbm.at[pl.ds(nb * FX, _ROWS_PER_CHUNK)], idxv)
        for j in range(_ROWS_PER_CHUNK // 16):
            idxv[pl.ds(16 * j, 16)] = idxv[pl.ds(16 * j, 16)] + coff
        pltpu.async_copy(atab_hbm.at[idxv], gbuf, sem).wait()

        def node(n, _):
            for j in range(HD // 16):
                acc = gbuf[n * FX, pl.ds(16 * j, 16)]
                for f in range(1, FX):
                    acc = acc + gbuf[n * FX + f, pl.ds(16 * j, 16)]
                obuf[n, pl.ds(16 * j, 16)] = acc
            return 0

        lax.fori_loop(0, _NODE_CHUNK, node, 0)
        pltpu.sync_copy(obuf, h_hbm.at[pl.ds(c * N + nb, _NODE_CHUNK)])
        return 0

    lax.fori_loop(0, nch, chunk, 0)


def _encode_sc(xid, atab_cat):
    mesh = plsc.VectorSubcoreMesh(core_axis_name="c", subcore_axis_name="s")
    return pl.kernel(
        _enc_body,
        out_type=jax.ShapeDtypeStruct((2 * N, HD), jnp.float32),
        mesh=mesh,
        scratch_types=[
            pltpu.VMEM((_ROWS_PER_CHUNK,), jnp.int32),
            pltpu.VMEM((_ROWS_PER_CHUNK, HD), jnp.float32),
            pltpu.VMEM((_NODE_CHUNK, HD), jnp.float32),
            pltpu.SemaphoreType.DMA,
        ],
    )(xid, atab_cat)


_ECH = 80
_NCHUNK = E // NSUB // _ECH
_ZROWS = 80


def _msg_body(h_hbm, ctab_hbm, eidx_hbm, agg_hbm,
              pidx0, pidx1, dbuf0, dbuf1, hbuf0, hbuf1, cbuf0, cbuf1, accum,
              semi0, semi1, semh0, semh1, semc0, semc1, sems0, sems1):
    c = lax.axis_index("c")
    s = lax.axis_index("s")
    cN = c * N
    c512 = c * 512
    tbase = s * _NCHUNK * 3 * _ECH

    bufs = ((pidx0, dbuf0, hbuf0, cbuf0, semi0, semh0, semc0, sems0),
            (pidx1, dbuf1, hbuf1, cbuf1, semi1, semh1, semc1, sems1))

    def issue_idx(k, slot):
        pi, _, _, _, si, _, _, _ = bufs[slot]
        pltpu.async_copy(
            eidx_hbm.at[pl.ds(tbase + k * 3 * _ECH, 3 * _ECH)], pi, si)

    def wait_idx(slot):
        pi, _, _, _, si, _, _, _ = bufs[slot]
        pltpu.make_async_copy(eidx_hbm.at[pl.ds(0, 3 * _ECH)], pi, si).wait()

    def prep_and_gather(slot):
        pi, db, hb, cb, _, sh, sc_, _ = bufs[slot]
        for j in range(_ECH // 16):
            pi[pl.ds(16 * j, 16)] = pi[pl.ds(16 * j, 16)] + cN
            pi[pl.ds(_ECH + 16 * j, 16)] = pi[pl.ds(_ECH + 16 * j, 16)] + c512
            db[pl.ds(16 * j, 16)] = pi[pl.ds(2 * _ECH + 16 * j, 16)]
        pltpu.async_copy(h_hbm.at[pi.at[pl.ds(0, _ECH)]], hb, sh)
        pltpu.async_copy(ctab_hbm.at[pi.at[pl.ds(_ECH, _ECH)]], cb, sc_)

    def wait_gathers(slot):
        pi, _, hb, cb, _, sh, sc_, _ = bufs[slot]
        pltpu.make_async_copy(h_hbm.at[pi.at[pl.ds(0, _ECH)]], hb, sh).wait()
        pltpu.make_async_copy(
            ctab_hbm.at[pi.at[pl.ds(_ECH, _ECH)]], cb, sc_).wait()

    def wait_scatter(slot):
        _, db, hb, _, _, _, _, ss = bufs[slot]
        pltpu.make_async_copy(hb, accum.at[db], ss).wait()

    rbase = s * 640
    nz = jnp.where(s < NSUB - 1, 8, 5)

    issue_idx(0, 0)

    def zrow(r, _):
        for j in range(HD // 16):
            hbuf0[r, pl.ds(16 * j, 16)] = jnp.zeros((16,), jnp.float32)
        return 0

    lax.fori_loop(0, _ZROWS, zrow, 0)

    def zcopy(r, _):
        pltpu.sync_copy(hbuf0, accum.at[pl.ds(rbase + r * _ZROWS, _ZROWS)])
        return 0

    lax.fori_loop(0, nz, zcopy, 0)

    wait_idx(0)
    prep_and_gather(0)
    issue_idx(1, 1)
    plsc.subcore_barrier()

    def do_chunk(k, slot):
        pi, db, hb, cb, _, _, _, ss = bufs[slot]
        nxt = 1 - slot
        wait_gathers(slot)

        @pl.when(k + 1 < _NCHUNK)
        def _():
            wait_idx(nxt)

            @pl.when(k >= 1)
            def _():
                wait_scatter(nxt)
            prep_and_gather(nxt)

        @pl.when(k + 2 < _NCHUNK)
        def _():
            issue_idx(k + 2, slot)

        def edge(e, _):
            for j in range(HD // 16):
                hv = hb[e, pl.ds(16 * j, 16)]
                cv = cb[e, pl.ds(16 * j, 16)]
                hb[e, pl.ds(16 * j, 16)] = jnp.maximum(hv + cv, 0.0)
            return 0

        lax.fori_loop(0, _ECH, edge, 0)
        pltpu.async_copy(hb, accum.at[db], ss, add=True)

    def chunk(k, _):
        @pl.when(k % 2 == 0)
        def _():
            do_chunk(k, 0)

        @pl.when(k % 2 == 1)
        def _():
            do_chunk(k, 1)
        return 0

    lax.fori_loop(0, _NCHUNK, chunk, 0)
    wait_scatter((_NCHUNK - 2) % 2)
    wait_scatter((_NCHUNK - 1) % 2)
    plsc.subcore_barrier()

    def wcopy(r, _):
        pltpu.sync_copy(accum.at[pl.ds(rbase + r * _ZROWS, _ZROWS)],
                        agg_hbm.at[pl.ds(c * N + rbase + r * _ZROWS, _ZROWS)])
        return 0

    lax.fori_loop(0, nz, wcopy, 0)


def _msg_sc(h_cat, ctab_cat, eidx):
    mesh = plsc.VectorSubcoreMesh(core_axis_name="c", subcore_axis_name="s")
    return pl.kernel(
        _msg_body,
        out_type=jax.ShapeDtypeStruct((2 * N, HD), jnp.float32),
        mesh=mesh,
        scratch_types=[
            pltpu.VMEM((3 * _ECH,), jnp.int32),
            pltpu.VMEM((3 * _ECH,), jnp.int32),
            pltpu.VMEM((_ECH,), jnp.int32),
            pltpu.VMEM((_ECH,), jnp.int32),
            pltpu.VMEM((_ECH, HD), jnp.float32),
            pltpu.VMEM((_ECH, HD), jnp.float32),
            pltpu.VMEM((_ECH, HD), jnp.float32),
            pltpu.VMEM((_ECH, HD), jnp.float32),
            pltpu.VMEM_SHARED((N, HD), jnp.float32),
        ] + [pltpu.SemaphoreType.DMA] * 8,
    )(h_cat, ctab_cat, eidx)


def _ctab_body(be_ref, out_ref):
    rr = lax.broadcasted_iota(jnp.int32, (512, 8), 0)
    cc = lax.broadcasted_iota(jnp.int32, (512, 8), 1)
    a0 = ((rr // 64) == cc).astype(jnp.float32)
    a1 = (((rr // 8) % 8) == cc).astype(jnp.float32)
    a2 = ((rr % 8) == cc).astype(jnp.float32)
    c = (jnp.dot(a0, be_ref[0], preferred_element_type=jnp.float32, precision=lax.Precision.HIGHEST)
         + jnp.dot(a1, be_ref[1], preferred_element_type=jnp.float32, precision=lax.Precision.HIGHEST)
         + jnp.dot(a2, be_ref[2], preferred_element_type=jnp.float32, precision=lax.Precision.HIGHEST))
    out_ref[...] = jnp.concatenate([c[:, :HD], c[:, HD:]], axis=0)


def _ctab_tc(bond_emb):
    return pl.pallas_call(
        _ctab_body,
        out_shape=jax.ShapeDtypeStruct((1024, HD), jnp.float32),
    )(bond_emb)


def _phase_a_body(h0_ref, h1_ref, a0_ref, a1_ref, gw_ref, w1_ref, w2_ref,
                  batch_ref, hn_ref, psum_ref, cnts_ref, acc_ref):
    i = pl.program_id(0)
    z = jnp.concatenate(
        [h0_ref[...] + a0_ref[...], h1_ref[...] + a1_ref[...]], axis=1)
    zb = z.astype(jnp.bfloat16)
    logits = jnp.dot(zb, gw_ref[...], preferred_element_type=jnp.float32)
    lane = lax.broadcasted_iota(jnp.int32, (1, 128), 1)
    neg = jnp.float32(-1e30)
    logits = logits + jnp.where(lane < NEXP, 0.0, neg)
    v1 = jnp.max(logits, axis=1, keepdims=True)
    i1 = jnp.min(jnp.where(logits == v1, lane, 10 ** 9), axis=1, keepdims=True)
    l2 = jnp.where(lane == i1, neg, logits)
    v2 = jnp.max(l2, axis=1, keepdims=True)
    i2 = jnp.min(jnp.where(l2 == v2, lane, 10 ** 9), axis=1, keepdims=True)
    e2 = jnp.exp(v2 - v1)
    p1 = 1.0 / (1.0 + e2)
    p2 = e2 * p1
    gates = jnp.where(lane == i1, p1, 0.0) + jnp.where(lane == i2, p2, 0.0)
    ex = jnp.exp(logits - v1)
    sm = ex / jnp.sum(ex, axis=1, keepdims=True)

    h1 = jnp.maximum(
        jnp.dot(zb, w1_ref[...], preferred_element_type=jnp.float32), 0.0)
    hn = jnp.zeros((BR, D), jnp.float32)
    for e in range(NEXP):
        h2e = jnp.dot(h1[:, e * H:(e + 1) * H].astype(jnp.bfloat16),
                      w2_ref[e * H:(e + 1) * H, :],
                      preferred_element_type=jnp.float32)
        ge = gates[:, e:e + 1].astype(jnp.bfloat16).astype(jnp.float32)
        hn = hn + ge * h2e.astype(jnp.bfloat16).astype(jnp.float32)
    hn_ref[...] = hn

    b = batch_ref[0, 0]
    onehot = (b[:, None] == lax.broadcasted_iota(jnp.int32, (BR, NG), 1)
              ).astype(jnp.float32)
    ps = lax.dot_general(onehot, hn, (((0,), (0,)), ((), ())),
                         preferred_element_type=jnp.float32, precision=lax.Precision.HIGHEST)
    cnt = lax.dot_general(onehot, jnp.ones((BR, 8), jnp.float32),
                          (((0,), (0,)), ((), ())),
                          preferred_element_type=jnp.float32, precision=lax.Precision.HIGHEST)
    pad = jnp.zeros((1, 128), jnp.float32)
    acc = jnp.concatenate([
        jnp.sum(hn, axis=0, keepdims=True),
        jnp.sum(hn * hn, axis=0, keepdims=True),
        jnp.concatenate([jnp.sum(gates, axis=0, keepdims=True), pad], axis=1),
        jnp.concatenate([jnp.sum(sm, axis=0, keepdims=True), pad], axis=1),
        jnp.zeros((4, 256), jnp.float32),
    ], axis=0)

    @pl.when(i == 0)
    def _():
        psum_ref[...] = ps
        cnts_ref[...] = cnt
        acc_ref[...] = acc

    @pl.when(i > 0)
    def _():
        psum_ref[...] = psum_ref[...] + ps
        cnts_ref[...] = cnts_ref[...] + cnt
        acc_ref[...] = acc_ref[...] + acc


def _phase_a(h_cat, agg_cat, gw_pad, w1cat, w2cat, batch3):
    return pl.pallas_call(
        _phase_a_body,
        grid=(NB,),
        in_specs=[
            pl.BlockSpec((BR, HD), lambda i: (i, 0)),
            pl.BlockSpec((BR, HD), lambda i: (NB + i, 0)),
            pl.BlockSpec((BR, HD), lambda i: (i, 0)),
            pl.BlockSpec((BR, HD), lambda i: (NB + i, 0)),
            pl.BlockSpec((D, 128), lambda i: (0, 0)),
            pl.BlockSpec((D, NEXP * H), lambda i: (0, 0)),
            pl.BlockSpec((NEXP * H, D), lambda i: (0, 0)),
            pl.BlockSpec((1, 1, BR), lambda i: (i, 0, 0)),
        ],
        out_specs=[
            pl.BlockSpec((BR, D), lambda i: (i, 0)),
            pl.BlockSpec((NG, D), lambda i: (0, 0)),
            pl.BlockSpec((NG, 8), lambda i: (0, 0)),
            pl.BlockSpec((8, D), lambda i: (0, 0)),
        ],
        out_shape=[
            jax.ShapeDtypeStruct((N, D), jnp.float32),
            jax.ShapeDtypeStruct((NG, D), jnp.float32),
            jax.ShapeDtypeStruct((NG, 8), jnp.float32),
            jax.ShapeDtypeStruct((8, D), jnp.float32),
        ],
    )(h_cat, h_cat, agg_cat, agg_cat, gw_pad, w1cat, w2cat, batch3)


def _phase_b_body(hn_ref, acc_ref, g_ref, b_ref, out_ref):
    mu = acc_ref[0:1, :] / N
    var = acc_ref[1:2, :] / N - mu * mu
    scale = g_ref[0] * lax.rsqrt(var + 1e-5)
    shift = b_ref[0] - mu * scale
    out_ref[...] = jnp.maximum(hn_ref[...] * scale + shift, 0.0)


def _phase_b(hn, acc, bn_g2, bn_b2):
    return pl.pallas_call(
        _phase_b_body,
        grid=(2, NB),
        in_specs=[
            pl.BlockSpec((BR, HD), lambda h, j: (j, h)),
            pl.BlockSpec((8, HD), lambda h, j: (0, h)),
            pl.BlockSpec((1, 1, HD), lambda h, j: (h, 0, 0)),
            pl.BlockSpec((1, 1, HD), lambda h, j: (h, 0, 0)),
        ],
        out_specs=pl.BlockSpec((BR, HD), lambda h, j: (h * NB + j, 0)),
        out_shape=jax.ShapeDtypeStruct((2 * N, HD), jnp.float32),
    )(hn, acc, bn_g2, bn_b2)


def _final_body(ps0_ref, ps1_ref, ps2_ref, cnts_ref, ac0_ref, ac1_ref, ac2_ref,
                pw_ref, pb_ref, ow_ref, ob_ref, y_ref, aux_ref):
    cnt = jnp.maximum(cnts_ref[:, 0:1], 1.0)
    ys = []
    for l, ps_ref in enumerate((ps0_ref, ps1_ref, ps2_ref)):
        pooled = (ps_ref[...] / cnt).astype(jnp.bfloat16)
        ys.append(jnp.dot(pooled, pw_ref[l], preferred_element_type=jnp.float32)
                  + pb_ref[l])
    aux = jnp.float32(0.0)
    for ac_ref in (ac0_ref, ac1_ref, ac2_ref):
        aux = aux + jnp.sum(ac_ref[2:3, :] * ac_ref[3:4, :])
    aux = aux * (NEXP / float(N * N))
    ysb = [y.astype(jnp.bfloat16).astype(jnp.float32) for y in ys]
    owf = ow_ref[...].astype(jnp.float32)
    for u in range(T):
        yu = (ysb[0] * owf[u:u + 1, 0:1] + ysb[1] * owf[u:u + 1, 1:2]
              + ysb[2] * owf[u:u + 1, 2:3] + ob_ref[0:1, u:u + 1])
        y_ref[u] = 1.0 / (1.0 + jnp.exp(-yu))
    aux_ref[...] = jnp.full((8, 128), aux, jnp.float32)


def _final(psums, cnts, accs, pred_w, pred_b3, out_w, out_b2):
    return pl.pallas_call(
        _final_body,
        out_shape=[
            jax.ShapeDtypeStruct((T, NG, T), jnp.float32),
            jax.ShapeDtypeStruct((8, 128), jnp.float32),
        ],
    )(psums[0], psums[1], psums[2], cnts, accs[0], accs[1], accs[2],
      pred_w, pred_b3, out_w, out_b2)


def kernel(x, edge_index, edge_attr, batch, atom_emb, bond_emb, gate_w, w1,
           w2, bn_g, bn_b, pred_w, pred_b, out_w, out_b):
    x = x.astype(jnp.int32)
    edge_attr = edge_attr.astype(jnp.int32)
    src = edge_index[0].astype(jnp.int32)
    dst = edge_index[1].astype(jnp.int32)
    xid = (x + jnp.arange(FX, dtype=jnp.int32)[None, :] * 100).reshape(-1)
    cidx = edge_attr[:, 0] * 64 + edge_attr[:, 1] * 8 + edge_attr[:, 2]
    eidx = jnp.stack([src.reshape(-1, _ECH), cidx.reshape(-1, _ECH),
                      dst.reshape(-1, _ECH)], axis=1).reshape(-1)
    atab_cat = jnp.transpose(
        atom_emb.reshape(FX * 100, 2, HD), (1, 0, 2)).reshape(2 * FX * 100, HD)
    batch3 = batch.astype(jnp.int32).reshape(NB, 1, BR)
    gw_pad = jnp.concatenate(
        [gate_w, jnp.zeros((L, D, 128 - NEXP), jnp.float32)],
        axis=2).astype(jnp.bfloat16)
    w1cat = jnp.transpose(w1, (0, 2, 1, 3)).reshape(L, D, NEXP * H).astype(jnp.bfloat16)
    w2cat = w2.reshape(L, NEXP * H, D).astype(jnp.bfloat16)

    ctab_cat = _ctab_tc(bond_emb)
    h_cat = _encode_sc(xid, atab_cat)

    psums, accs = [], []
    cnts = None
    for l in range(L):
        agg_cat = _msg_sc(h_cat, ctab_cat, eidx)
        hn, psum, cnt, acc = _phase_a(
            h_cat, agg_cat, gw_pad[l], w1cat[l], w2cat[l], batch3)
        psums.append(psum)
        accs.append(acc)
        if cnts is None:
            cnts = cnt
        if l < L - 1:
            h_cat = _phase_b(hn, acc, bn_g[l].reshape(2, 1, HD),
                             bn_b[l].reshape(2, 1, HD))

    y3, aux = _final(psums, cnts, accs, pred_w.astype(jnp.bfloat16),
                     pred_b.reshape(L, 1, T), out_w.astype(jnp.bfloat16),
                     out_b.reshape(1, T))
    y = jnp.transpose(y3, (1, 2, 0))
    return y, aux[0, 0]
